# baseline (speedup 1.0000x reference)
import functools

import jax
import jax.numpy as jnp
from jax import lax
from jax.experimental import pallas as pl
from jax.experimental.pallas import tpu as pltpu
from jax.experimental.pallas import tpu_sc as plsc

NPART = 5000
NBEAM = 90
NBP = 96
NV = NBP // 16
NCORES = 2
NSUB = 16
NTILES = NCORES * NSUB
PPT = 160
NPPAD = PPT * NTILES
NMETA = NPPAD + NTILES
MAPN = 800
L1MAX = 9
BLK = 5
W1_R = 2 * L1MAX + 3
W1_C = 32
WIN_R = 104
WIN_C = 112
NSTEP = 51
THRESH = 0.35
MAXRP = 50.0
MAGIC = 12582912.0


def _sc_raycast(occ, laser, cosb, sinb):
    mesh = plsc.VectorSubcoreMesh(core_axis_name="c", subcore_axis_name="s")

    @functools.partial(
        pl.kernel,
        out_type=jax.ShapeDtypeStruct((NPPAD, NBP), jnp.float32),
        mesh=mesh,
        scratch_types=[
            pltpu.VMEM((PPT + 2, 16), jnp.float32),
            pltpu.VMEM((PPT, NBP), jnp.float32),
            pltpu.VMEM((PPT, NBP), jnp.float32),
            pltpu.VMEM((PPT, NBP), jnp.float32),
            pltpu.VMEM((W1_R, W1_C), jnp.float32),
            pltpu.VMEM((W1_R, W1_C), jnp.float32),
            pltpu.VMEM((W1_R, W1_C), jnp.float32),
            pltpu.VMEM((W1_R, W1_C), jnp.float32),
            pltpu.VMEM((WIN_R, WIN_C), jnp.float32),
            pltpu.SemaphoreType.DMA,
            pltpu.SemaphoreType.DMA,
            pltpu.SemaphoreType.DMA,
            pltpu.SemaphoreType.DMA,
        ],
        compiler_params=pltpu.CompilerParams(
            use_tc_tiling_on_sc=False, needs_layout_passes=False,
            disable_bounds_checks=True),
    )
    def k(occ_hbm, las_hbm, cos_hbm, sin_hbm, out_hbm,
          las_v, cos_v, sin_v, acc_v, win_0, win_1, win_2, win_3, win2_v,
          sem_0, sem_1, sem_2, sem_3):
        bufs = (win_0, win_1, win_2, win_3)
        sems = (sem_0, sem_1, sem_2, sem_3)
        wid = lax.axis_index("c") * NSUB + lax.axis_index("s")
        p0 = wid * PPT
        pltpu.sync_copy(las_hbm.at[pl.ds(p0, PPT + 2), :], las_v)
        pltpu.sync_copy(cos_hbm.at[pl.ds(p0, PPT), :], cos_v)
        pltpu.sync_copy(sin_hbm.at[pl.ds(p0, PPT), :], sin_v)

        def n_unresolved(bhl):
            return plsc.all_reduce_population_count(bhl >= MAXRP)[0]

        def round_i(x):
            return ((x + MAGIC) - MAGIC).astype(jnp.int32)

        def clamp(x, lo, hi):
            return jnp.minimum(jnp.maximum(x, lo), hi)

        def win1_origin(p):
            row = las_v[p, :]
            riy = round_i(row[1])
            rix = round_i(row[0])
            r1 = clamp(riy - (L1MAX + 1), 0, MAPN - W1_R)
            c1 = (clamp(rix - (L1MAX + 1), 0, MAPN - W1_C) >> 3) << 3
            return r1, c1

        def issue_win1(p, buf, sem):
            r1, c1 = win1_origin(p)
            return pltpu.async_copy(
                occ_hbm.at[pl.ds(r1, W1_R),
                           pl.ds(pl.multiple_of(c1, 8), W1_C)], buf, sem)

        def process(p, win1_v):
            row = las_v[p, :]
            xl = row[0]
            yl = row[1]
            r1, c1 = win1_origin(p)

            cs = [cos_v[p, pl.ds(v * 16, 16)] for v in range(NV)]
            ss = [sin_v[p, pl.ds(v * 16, 16)] for v in range(NV)]

            def fast_cond(carry):
                L = carry[0]
                bhls = carry[1:]
                m = bhls[0]
                for b in bhls[1:]:
                    m = jnp.maximum(m, b)
                return (L < L1MAX + 1) & (n_unresolved(m) > 0)

            def fast_blk(carry):
                L = carry[0]
                bhls = list(carry[1:])
                for kk in range(BLK):
                    lf = lax.convert_element_type(L + kk, jnp.float32)
                    for v in range(NV):
                        xf = xl + cs[v] * lf
                        yf = yl + ss[v] * lf
                        xi = round_i(xf)
                        yi = round_i(yf)
                        val = plsc.load_gather(win1_v, [yi - r1, xi - c1])
                        bhls[v] = jnp.minimum(
                            bhls[v], jnp.where(val > THRESH, lf, MAXRP))
                return (L + BLK, *bhls)

            bhl0 = jnp.full((16,), MAXRP, dtype=jnp.float32)
            res = lax.while_loop(fast_cond, fast_blk, (0,) + (bhl0,) * NV)
            bhls = res[1:]
            mx = bhls[0]
            for b in bhls[1:]:
                mx = jnp.maximum(mx, b)
            total_unres = n_unresolved(mx)
            for v in range(NV):
                acc_v[p, pl.ds(v * 16, 16)] = bhls[v]

            @pl.when(total_unres > 0)
            def _fallback():
                riy = round_i(yl)
                rix = round_i(xl)
                row_lo = clamp(riy - 51, 0, MAPN - WIN_R)
                col_lo = (clamp(rix - 51, 0, MAPN - WIN_C) >> 3) << 3
                pltpu.sync_copy(
                    occ_hbm.at[pl.ds(row_lo, WIN_R),
                               pl.ds(pl.multiple_of(col_lo, 8), WIN_C)],
                    win2_v)

                for v in range(NV):
                    c = cos_v[p, pl.ds(v * 16, 16)]
                    s = sin_v[p, pl.ds(v * 16, 16)]

                    def slow_cond(carry):
                        L, bhl = carry
                        return (L < NSTEP) & (n_unresolved(bhl) > 0)

                    def slow_step(carry):
                        L, bhl = carry
                        lf = lax.convert_element_type(L, jnp.float32)
                        xf = xl + c * lf
                        yf = yl + s * lf
                        xi = clamp(round_i(xf), 0, MAPN - 1)
                        yi = clamp(round_i(yf), 0, MAPN - 1)
                        val = plsc.load_gather(
                            win2_v, [yi - row_lo, xi - col_lo])
                        hit = jnp.where(val > THRESH, lf, MAXRP)
                        return L + 1, jnp.minimum(bhl, hit)

                    bhl0 = acc_v[p, pl.ds(v * 16, 16)]
                    _, bhl = lax.while_loop(
                        slow_cond, slow_step, (L1MAX + 1, bhl0))
                    acc_v[p, pl.ds(v * 16, 16)] = bhl

        for j in range(3):
            issue_win1(j, bufs[j], sems[j])

        def quad(i, carry):
            q = i * 4
            for j in range(4):
                p = q + j
                nb = (j + 3) % 4

                @pl.when(p + 3 < PPT)
                def _issue_ahead():
                    issue_win1(p + 3, bufs[nb], sems[nb])

                pltpu.make_async_copy(
                    occ_hbm.at[pl.ds(0, W1_R), pl.ds(0, W1_C)], bufs[j],
                    sems[j]).wait()
                process(p, bufs[j])
            return carry

        lax.fori_loop(0, PPT // 4, quad, 0)
        pltpu.sync_copy(acc_v, out_hbm.at[pl.ds(p0, PPT), :])

    return k(occ, laser, cosb, sinb)


def kernel(X_t1, occupancy_map):
    f32 = jnp.float32
    xb, yb, yaw = X_t1[:, 0], X_t1[:, 1], X_t1[:, 2]
    xl = (xb + 25.0 * jnp.cos(yaw)) / 10.0
    yl = (yb + 25.0 * jnp.sin(yaw)) / 10.0

    las2 = jnp.stack([xl, yl], axis=1)
    laser = jnp.pad(las2, ((0, NMETA - NPART), (0, 14)),
                    constant_values=400.0)

    angles = jnp.arange(-90, 90, 180 // NBEAM).astype(f32)
    angp = jnp.pad(jnp.deg2rad(angles), (0, NBP - NBEAM))
    yawp = jnp.pad(yaw, (0, NPPAD - NPART))
    beam_angle = angp[None, :] + yawp[:, None]
    cosp = jnp.cos(beam_angle)
    sinp = jnp.sin(beam_angle)

    out = _sc_raycast(occupancy_map, laser, cosp, sinp)
    return out[:NPART, :NBEAM]


# speedup vs baseline: 1.0407x; 1.0407x over previous
import functools

import jax
import jax.numpy as jnp
from jax import lax
from jax.experimental import pallas as pl
from jax.experimental.pallas import tpu as pltpu
from jax.experimental.pallas import tpu_sc as plsc

NPART = 5000
NBEAM = 90
NBP = 96
NV = NBP // 16
NCORES = 2
NSUB = 16
NTILES = NCORES * NSUB
PPT = 160
NPPAD = PPT * NTILES
NMETA = NPPAD + NTILES
MAPN = 800
L1MAX = 14
BLK = 5
W1_R = 2 * L1MAX + 3
W1_C = 40
WIN_R = 104
WIN_C = 112
NSTEP = 51
THRESH = 0.35
MAXRP = 50.0
MAGIC = 12582912.0


def _sc_raycast(occ, laser, cosb, sinb):
    mesh = plsc.VectorSubcoreMesh(core_axis_name="c", subcore_axis_name="s")

    @functools.partial(
        pl.kernel,
        out_type=jax.ShapeDtypeStruct((NPPAD, NBP), jnp.float32),
        mesh=mesh,
        scratch_types=[
            pltpu.VMEM((PPT + 2, 16), jnp.float32),
            pltpu.VMEM((PPT, NBP), jnp.float32),
            pltpu.VMEM((PPT, NBP), jnp.float32),
            pltpu.VMEM((PPT, NBP), jnp.float32),
            pltpu.VMEM((W1_R, W1_C), jnp.float32),
            pltpu.VMEM((W1_R, W1_C), jnp.float32),
            pltpu.VMEM((W1_R, W1_C), jnp.float32),
            pltpu.VMEM((W1_R, W1_C), jnp.float32),
            pltpu.VMEM((WIN_R, WIN_C), jnp.float32),
            pltpu.SemaphoreType.DMA,
            pltpu.SemaphoreType.DMA,
            pltpu.SemaphoreType.DMA,
            pltpu.SemaphoreType.DMA,
        ],
        compiler_params=pltpu.CompilerParams(
            use_tc_tiling_on_sc=False, needs_layout_passes=False,
            disable_bounds_checks=True),
    )
    def k(occ_hbm, las_hbm, cos_hbm, sin_hbm, out_hbm,
          las_v, cos_v, sin_v, acc_v, win_0, win_1, win_2, win_3, win2_v,
          sem_0, sem_1, sem_2, sem_3):
        bufs = (win_0, win_1, win_2, win_3)
        sems = (sem_0, sem_1, sem_2, sem_3)
        wid = lax.axis_index("c") * NSUB + lax.axis_index("s")
        p0 = wid * PPT
        pltpu.sync_copy(las_hbm.at[pl.ds(p0, PPT + 2), :], las_v)
        pltpu.sync_copy(cos_hbm.at[pl.ds(p0, PPT), :], cos_v)
        pltpu.sync_copy(sin_hbm.at[pl.ds(p0, PPT), :], sin_v)

        def n_unresolved(bhl):
            return plsc.all_reduce_population_count(bhl >= MAXRP)[0]

        def round_i(x):
            return ((x + MAGIC) - MAGIC).astype(jnp.int32)

        def clamp(x, lo, hi):
            return jnp.minimum(jnp.maximum(x, lo), hi)

        def win1_origin(p):
            row = las_v[p, :]
            riy = round_i(row[1])
            rix = round_i(row[0])
            r1 = clamp(riy - (L1MAX + 1), 0, MAPN - W1_R)
            c1 = (clamp(rix - (L1MAX + 1), 0, MAPN - W1_C) >> 3) << 3
            return r1, c1

        def issue_win1(p, buf, sem):
            r1, c1 = win1_origin(p)
            return pltpu.async_copy(
                occ_hbm.at[pl.ds(r1, W1_R),
                           pl.ds(pl.multiple_of(c1, 8), W1_C)], buf, sem)

        def process(p, win1_v):
            row = las_v[p, :]
            xl = row[0]
            yl = row[1]
            r1, c1 = win1_origin(p)

            cs = [cos_v[p, pl.ds(v * 16, 16)] for v in range(NV)]
            ss = [sin_v[p, pl.ds(v * 16, 16)] for v in range(NV)]

            def fast_cond(carry):
                L = carry[0]
                bhls = carry[1:]
                m = bhls[0]
                for b in bhls[1:]:
                    m = jnp.maximum(m, b)
                return (L < L1MAX + 1) & (n_unresolved(m) > 0)

            def fast_blk(carry):
                L = carry[0]
                bhls = list(carry[1:])
                for kk in range(BLK):
                    lf = lax.convert_element_type(L + kk, jnp.float32)
                    for v in range(NV):
                        xf = xl + cs[v] * lf
                        yf = yl + ss[v] * lf
                        xi = round_i(xf)
                        yi = round_i(yf)
                        val = plsc.load_gather(win1_v, [yi - r1, xi - c1])
                        bhls[v] = jnp.minimum(
                            bhls[v], jnp.where(val > THRESH, lf, MAXRP))
                return (L + BLK, *bhls)

            bhl0 = jnp.full((16,), MAXRP, dtype=jnp.float32)
            res = lax.while_loop(fast_cond, fast_blk, (0,) + (bhl0,) * NV)
            bhls = res[1:]
            mx = bhls[0]
            for b in bhls[1:]:
                mx = jnp.maximum(mx, b)
            total_unres = n_unresolved(mx)
            for v in range(NV):
                acc_v[p, pl.ds(v * 16, 16)] = bhls[v]

            @pl.when(total_unres > 0)
            def _fallback():
                riy = round_i(yl)
                rix = round_i(xl)
                row_lo = clamp(riy - 51, 0, MAPN - WIN_R)
                col_lo = (clamp(rix - 51, 0, MAPN - WIN_C) >> 3) << 3
                pltpu.sync_copy(
                    occ_hbm.at[pl.ds(row_lo, WIN_R),
                               pl.ds(pl.multiple_of(col_lo, 8), WIN_C)],
                    win2_v)

                for v in range(NV):
                    c = cos_v[p, pl.ds(v * 16, 16)]
                    s = sin_v[p, pl.ds(v * 16, 16)]

                    def slow_cond(carry):
                        L, bhl = carry
                        return (L < NSTEP) & (n_unresolved(bhl) > 0)

                    def slow_step(carry):
                        L, bhl = carry
                        lf = lax.convert_element_type(L, jnp.float32)
                        xf = xl + c * lf
                        yf = yl + s * lf
                        xi = clamp(round_i(xf), 0, MAPN - 1)
                        yi = clamp(round_i(yf), 0, MAPN - 1)
                        val = plsc.load_gather(
                            win2_v, [yi - row_lo, xi - col_lo])
                        hit = jnp.where(val > THRESH, lf, MAXRP)
                        return L + 1, jnp.minimum(bhl, hit)

                    bhl0 = acc_v[p, pl.ds(v * 16, 16)]
                    _, bhl = lax.while_loop(
                        slow_cond, slow_step, (L1MAX + 1, bhl0))
                    acc_v[p, pl.ds(v * 16, 16)] = bhl

        for j in range(3):
            issue_win1(j, bufs[j], sems[j])

        def quad(i, carry):
            q = i * 4
            for j in range(4):
                p = q + j
                nb = (j + 3) % 4

                @pl.when(p + 3 < PPT)
                def _issue_ahead():
                    issue_win1(p + 3, bufs[nb], sems[nb])

                pltpu.make_async_copy(
                    occ_hbm.at[pl.ds(0, W1_R), pl.ds(0, W1_C)], bufs[j],
                    sems[j]).wait()
                process(p, bufs[j])
            return carry

        lax.fori_loop(0, PPT // 4, quad, 0)
        pltpu.sync_copy(acc_v, out_hbm.at[pl.ds(p0, PPT), :])

    return k(occ, laser, cosb, sinb)


def kernel(X_t1, occupancy_map):
    f32 = jnp.float32
    xb, yb, yaw = X_t1[:, 0], X_t1[:, 1], X_t1[:, 2]
    xl = (xb + 25.0 * jnp.cos(yaw)) / 10.0
    yl = (yb + 25.0 * jnp.sin(yaw)) / 10.0

    las2 = jnp.stack([xl, yl], axis=1)
    laser = jnp.pad(las2, ((0, NMETA - NPART), (0, 14)),
                    constant_values=400.0)

    angles = jnp.arange(-90, 90, 180 // NBEAM).astype(f32)
    angp = jnp.pad(jnp.deg2rad(angles), (0, NBP - NBEAM))
    yawp = jnp.pad(yaw, (0, NPPAD - NPART))
    beam_angle = angp[None, :] + yawp[:, None]
    cosp = jnp.cos(beam_angle)
    sinp = jnp.sin(beam_angle)

    out = _sc_raycast(occupancy_map, laser, cosp, sinp)
    return out[:NPART, :NBEAM]


# speedup vs baseline: 1.6386x; 1.5745x over previous
import functools

import jax
import jax.numpy as jnp
from jax import lax
from jax.experimental import pallas as pl
from jax.experimental.pallas import tpu as pltpu
from jax.experimental.pallas import tpu_sc as plsc

NPART = 5000
NBEAM = 90
NBP = 96
NV = NBP // 16
NCORES = 2
NSUB = 16
NTILES = NCORES * NSUB
PPT = 160
NPPAD = PPT * NTILES
NMETA = NPPAD + NTILES
MAPN = 800
L1MAX = 14
BLK = 5
W1_R = 2 * L1MAX + 3
W1_C = 40
WIN_R = 104
WIN_C = 112
NSTEP = 51
THRESH = 0.35
MAXRP = 50.0
MAGIC = 12582912.0


def _sc_raycast(occ, laser, cosb, sinb):
    mesh = plsc.VectorSubcoreMesh(core_axis_name="c", subcore_axis_name="s")

    @functools.partial(
        pl.kernel,
        out_type=jax.ShapeDtypeStruct((NPPAD, NBP), jnp.float32),
        mesh=mesh,
        scratch_types=[
            pltpu.VMEM((PPT + 2, 16), jnp.float32),
            pltpu.VMEM((PPT, NBP), jnp.float32),
            pltpu.VMEM((PPT, NBP), jnp.float32),
            pltpu.VMEM((PPT, NBP), jnp.float32),
            pltpu.VMEM((4 * W1_R, W1_C), jnp.float32),
            pltpu.VMEM((WIN_R, WIN_C), jnp.float32),
            pltpu.SemaphoreType.DMA,
            pltpu.SemaphoreType.DMA,
            pltpu.SemaphoreType.DMA,
            pltpu.SemaphoreType.DMA,
        ],
        compiler_params=pltpu.CompilerParams(
            use_tc_tiling_on_sc=False, needs_layout_passes=False,
            disable_bounds_checks=True),
    )
    def k(occ_hbm, las_hbm, cos_hbm, sin_hbm, out_hbm,
          las_v, cos_v, sin_v, acc_v, win_ring, win2_v,
          sem_0, sem_1, sem_2, sem_3):
        sems = (sem_0, sem_1, sem_2, sem_3)
        wid = lax.axis_index("c") * NSUB + lax.axis_index("s")
        p0 = wid * PPT
        pltpu.sync_copy(las_hbm.at[pl.ds(p0, PPT + 2), :], las_v)
        pltpu.sync_copy(cos_hbm.at[pl.ds(p0, PPT), :], cos_v)
        pltpu.sync_copy(sin_hbm.at[pl.ds(p0, PPT), :], sin_v)

        def n_unresolved(bhl):
            return plsc.all_reduce_population_count(bhl >= MAXRP)[0]

        def round_i(x):
            return ((x + MAGIC) - MAGIC).astype(jnp.int32)

        def clamp(x, lo, hi):
            return jnp.minimum(jnp.maximum(x, lo), hi)

        def win1_origin(p):
            row = las_v[p, :]
            riy = round_i(row[1])
            rix = round_i(row[0])
            r1 = clamp(riy - (L1MAX + 1), 0, MAPN - W1_R)
            c1 = (clamp(rix - (L1MAX + 1), 0, MAPN - W1_C) >> 3) << 3
            return r1, c1

        def issue_win1(p, slot):
            r1, c1 = win1_origin(p)
            return pltpu.async_copy(
                occ_hbm.at[pl.ds(r1, W1_R),
                           pl.ds(pl.multiple_of(c1, 8), W1_C)],
                win_ring.at[pl.ds(slot * W1_R, W1_R), :], sems[slot])

        def drain_slot(slot):
            pltpu.make_async_copy(
                occ_hbm.at[pl.ds(0, W1_R), pl.ds(0, W1_C)],
                win_ring.at[pl.ds(slot * W1_R, W1_R), :], sems[slot]).wait()

        def process(p, roff):
            win1_v = win_ring
            row = las_v[p, :]
            xl = row[0]
            yl = row[1]
            r1, c1 = win1_origin(p)
            r1 = r1 - roff

            cs = [cos_v[p, pl.ds(v * 16, 16)] for v in range(NV)]
            ss = [sin_v[p, pl.ds(v * 16, 16)] for v in range(NV)]

            def fast_cond(carry):
                L = carry[0]
                bhls = carry[1:]
                m = bhls[0]
                for b in bhls[1:]:
                    m = jnp.maximum(m, b)
                return (L < L1MAX + 1) & (n_unresolved(m) > 0)

            def fast_blk(carry):
                L = carry[0]
                bhls = list(carry[1:])
                for kk in range(BLK):
                    lf = lax.convert_element_type(L + kk, jnp.float32)
                    for v in range(NV):
                        xf = xl + cs[v] * lf
                        yf = yl + ss[v] * lf
                        xi = round_i(xf)
                        yi = round_i(yf)
                        val = plsc.load_gather(win1_v, [yi - r1, xi - c1])
                        bhls[v] = jnp.minimum(
                            bhls[v], jnp.where(val > THRESH, lf, MAXRP))
                return (L + BLK, *bhls)

            bhl0 = jnp.full((16,), MAXRP, dtype=jnp.float32)
            res = lax.while_loop(fast_cond, fast_blk, (0,) + (bhl0,) * NV)
            bhls = res[1:]
            mx = bhls[0]
            for b in bhls[1:]:
                mx = jnp.maximum(mx, b)
            total_unres = n_unresolved(mx)
            for v in range(NV):
                acc_v[p, pl.ds(v * 16, 16)] = bhls[v]

            @pl.when(total_unres > 0)
            def _fallback():
                riy = round_i(yl)
                rix = round_i(xl)
                row_lo = clamp(riy - 51, 0, MAPN - WIN_R)
                col_lo = (clamp(rix - 51, 0, MAPN - WIN_C) >> 3) << 3
                pltpu.sync_copy(
                    occ_hbm.at[pl.ds(row_lo, WIN_R),
                               pl.ds(pl.multiple_of(col_lo, 8), WIN_C)],
                    win2_v)

                for v in range(NV):
                    c = cos_v[p, pl.ds(v * 16, 16)]
                    s = sin_v[p, pl.ds(v * 16, 16)]

                    def slow_cond(carry):
                        L, bhl = carry
                        return (L < NSTEP) & (n_unresolved(bhl) > 0)

                    def slow_step(carry):
                        L, bhl = carry
                        lf = lax.convert_element_type(L, jnp.float32)
                        xf = xl + c * lf
                        yf = yl + s * lf
                        xi = clamp(round_i(xf), 0, MAPN - 1)
                        yi = clamp(round_i(yf), 0, MAPN - 1)
                        val = plsc.load_gather(
                            win2_v, [yi - row_lo, xi - col_lo])
                        hit = jnp.where(val > THRESH, lf, MAXRP)
                        return L + 1, jnp.minimum(bhl, hit)

                    bhl0 = acc_v[p, pl.ds(v * 16, 16)]
                    _, bhl = lax.while_loop(
                        slow_cond, slow_step, (L1MAX + 1, bhl0))
                    acc_v[p, pl.ds(v * 16, 16)] = bhl

        for j in range(4):
            issue_win1(j, j)

        def body(p, carry):
            jj = lax.rem(p, 4)
            for j in range(4):
                @pl.when(jj == j)
                def _drain():
                    drain_slot(j)
            process(p, jj * W1_R)

            @pl.when(p + 4 < PPT)
            def _refill():
                for j in range(4):
                    @pl.when(jj == j)
                    def _issue():
                        issue_win1(p + 4, j)

            return carry

        lax.fori_loop(0, PPT, body, 0)
        pltpu.sync_copy(acc_v, out_hbm.at[pl.ds(p0, PPT), :])

    return k(occ, laser, cosb, sinb)


def kernel(X_t1, occupancy_map):
    f32 = jnp.float32
    xb, yb, yaw = X_t1[:, 0], X_t1[:, 1], X_t1[:, 2]
    xl = (xb + 25.0 * jnp.cos(yaw)) / 10.0
    yl = (yb + 25.0 * jnp.sin(yaw)) / 10.0

    las2 = jnp.stack([xl, yl], axis=1)
    laser = jnp.pad(las2, ((0, NMETA - NPART), (0, 14)),
                    constant_values=400.0)

    angles = jnp.arange(-90, 90, 180 // NBEAM).astype(f32)
    angp = jnp.pad(jnp.deg2rad(angles), (0, NBP - NBEAM))
    yawp = jnp.pad(yaw, (0, NPPAD - NPART))
    beam_angle = angp[None, :] + yawp[:, None]
    cosp = jnp.cos(beam_angle)
    sinp = jnp.sin(beam_angle)

    out = _sc_raycast(occupancy_map, laser, cosp, sinp)
    return out[:NPART, :NBEAM]
